# baseline (speedup 1.0000x reference)
import functools

import jax
import jax.numpy as jnp
from jax import lax
from jax.experimental import pallas as pl
from jax.experimental.pallas import tpu as pltpu
from jax.experimental.pallas import tpu_sc as plsc

BATCH = 16384
K = 128
NW = 32
BPW = BATCH // NW
NCHUNK = 4
CHUNK = BPW // NCHUNK
GROUPS = CHUNK // 16


def _mf_body(user_hbm, item_hbm, gu_hbm, gi_hbm, bu_hbm, bi_hbm, mu_hbm,
             out_hbm, idx_u, idx_i, ru0, ru1, ri0, ri1, bu_v, bi_v, mu_v,
             out_v, sem0, sem1, semb):
    c = lax.axis_index("c")
    s = lax.axis_index("s")
    wid = s * 2 + c
    base = wid * BPW

    pltpu.sync_copy(mu_hbm, mu_v)
    for ch in range(NCHUNK):
        pltpu.sync_copy(user_hbm.at[pl.ds(base + ch * CHUNK, CHUNK)],
                        idx_u.at[ch])
        pltpu.sync_copy(item_hbm.at[pl.ds(base + ch * CHUNK, CHUNK)],
                        idx_i.at[ch])
    mu = mu_v[...]

    bias_copies = []
    for ch in range(NCHUNK):
        bias_copies.append(pltpu.async_copy(
            bu_hbm.at[idx_u.at[ch]], bu_v.at[pl.ds(ch * CHUNK, CHUNK)], semb))
        bias_copies.append(pltpu.async_copy(
            bi_hbm.at[idx_i.at[ch]], bi_v.at[pl.ds(ch * CHUNK, CHUNK)], semb))

    rbufs = [(ru0, ri0, sem0), (ru1, ri1, sem1)]

    def fire(ch):
        ru, ri, sem = rbufs[ch % 2]
        return (pltpu.async_copy(gu_hbm.at[idx_u.at[ch]], ru, sem),
                pltpu.async_copy(gi_hbm.at[idx_i.at[ch]], ri, sem))

    pending = fire(0)
    for ch in range(NCHUNK):
        nxt = fire(ch + 1) if ch + 1 < NCHUNK else None
        pending[0].wait()
        pending[1].wait()
        ru, ri, _ = rbufs[ch % 2]

        def group_body(g, carry, ru=ru, ri=ri, ch=ch):
            rid = g * 16 + lax.iota(jnp.int32, 16)
            acc = jnp.zeros((16,), jnp.float32)
            for k in range(K):
                col = jnp.full((16,), k, jnp.int32)
                uu = plsc.load_gather(ru, [rid, col])
                vv = plsc.load_gather(ri, [rid, col])
                acc = acc + uu * vv
            out_v[pl.ds(ch * CHUNK + g * 16, 16)] = acc
            return carry

        lax.fori_loop(0, GROUPS, group_body, 0)
        pending = nxt

    for cp in bias_copies:
        cp.wait()

    def epi_body(g, carry):
        sl = pl.ds(g * 16, 16)
        out_v[sl] = out_v[sl] + bu_v[sl] + bi_v[sl] + mu
        return carry

    lax.fori_loop(0, BPW // 16, epi_body, 0)
    pltpu.sync_copy(out_v, out_hbm.at[pl.ds(base, BPW)])


_mf_sc = functools.partial(
    pl.kernel,
    out_type=jax.ShapeDtypeStruct((BATCH,), jnp.float32),
    mesh=plsc.VectorSubcoreMesh(core_axis_name="c", subcore_axis_name="s"),
    compiler_params=pltpu.CompilerParams(needs_layout_passes=False),
    scratch_types=[
        pltpu.VMEM((NCHUNK, CHUNK), jnp.int32),
        pltpu.VMEM((NCHUNK, CHUNK), jnp.int32),
        pltpu.VMEM((CHUNK, K), jnp.float32),
        pltpu.VMEM((CHUNK, K), jnp.float32),
        pltpu.VMEM((CHUNK, K), jnp.float32),
        pltpu.VMEM((CHUNK, K), jnp.float32),
        pltpu.VMEM((BPW,), jnp.float32),
        pltpu.VMEM((BPW,), jnp.float32),
        pltpu.VMEM((16,), jnp.float32),
        pltpu.VMEM((BPW,), jnp.float32),
        pltpu.SemaphoreType.DMA,
        pltpu.SemaphoreType.DMA,
        pltpu.SemaphoreType.DMA,
    ],
)(_mf_body)


def kernel(user, item, Gu, Gi, Bu, Bi, Mu):
    mu16 = jnp.broadcast_to(jnp.reshape(Mu, (1,)), (16,))
    return _mf_sc(user.astype(jnp.int32), item.astype(jnp.int32), Gu, Gi,
                  jnp.reshape(Bu, (-1,)), jnp.reshape(Bi, (-1,)), mu16)


# speedup vs baseline: 1.3899x; 1.3899x over previous
import functools

import jax
import jax.numpy as jnp
from jax import lax
from jax.experimental import pallas as pl
from jax.experimental.pallas import tpu as pltpu
from jax.experimental.pallas import tpu_sc as plsc

BATCH = 16384
K = 128
NW = 32
BPW = BATCH // NW
NCHUNK = 4
CHUNK = BPW // NCHUNK
GROUPS = CHUNK // 16


def _mf_body(user_hbm, item_hbm, gu_hbm, gi_hbm, bu_hbm, bi_hbm, mu_hbm,
             out_hbm, idx_u, idx_i, ru0, ru1, ri0, ri1, bu_v, bi_v, mu_v,
             out_v, sem0, sem1, semb):
    c = lax.axis_index("c")
    s = lax.axis_index("s")
    wid = s * 2 + c
    base = wid * BPW

    pltpu.sync_copy(mu_hbm, mu_v)
    for ch in range(NCHUNK):
        pltpu.sync_copy(user_hbm.at[pl.ds(base + ch * CHUNK, CHUNK)],
                        idx_u.at[ch])
        pltpu.sync_copy(item_hbm.at[pl.ds(base + ch * CHUNK, CHUNK)],
                        idx_i.at[ch])
    mu = mu_v[...]

    bias_copies = []
    for ch in range(NCHUNK):
        bias_copies.append(pltpu.async_copy(
            bu_hbm.at[idx_u.at[ch]], bu_v.at[pl.ds(ch * CHUNK, CHUNK)], semb))
        bias_copies.append(pltpu.async_copy(
            bi_hbm.at[idx_i.at[ch]], bi_v.at[pl.ds(ch * CHUNK, CHUNK)], semb))

    rbufs = [(ru0, ri0, sem0), (ru1, ri1, sem1)]

    def fire(ch):
        ru, ri, sem = rbufs[ch % 2]
        return (pltpu.async_copy(gu_hbm.at[idx_u.at[ch]], ru, sem),
                pltpu.async_copy(gi_hbm.at[idx_i.at[ch]], ri, sem))

    pending = fire(0)
    for ch in range(NCHUNK):
        nxt = fire(ch + 1) if ch + 1 < NCHUNK else None
        pending[0].wait()
        pending[1].wait()
        ru, ri, _ = rbufs[ch % 2]

        def group_body(g, carry, ru=ru, ri=ri, ch=ch):
            lane = lax.iota(jnp.int32, 16)
            rid = g * 16 + lane
            accs = [jnp.zeros((16,), jnp.float32) for _ in range(4)]
            for k in range(K):
                col = (lane + k) & (K - 1)
                uu = plsc.load_gather(ru, [rid, col])
                vv = plsc.load_gather(ri, [rid, col])
                accs[k % 4] = accs[k % 4] + uu * vv
            acc = (accs[0] + accs[1]) + (accs[2] + accs[3])
            out_v[pl.ds(ch * CHUNK + g * 16, 16)] = acc
            return carry

        lax.fori_loop(0, GROUPS, group_body, 0)
        pending = nxt

    for cp in bias_copies:
        cp.wait()

    def epi_body(g, carry):
        sl = pl.ds(g * 16, 16)
        out_v[sl] = out_v[sl] + bu_v[sl] + bi_v[sl] + mu
        return carry

    lax.fori_loop(0, BPW // 16, epi_body, 0)
    pltpu.sync_copy(out_v, out_hbm.at[pl.ds(base, BPW)])


_mf_sc = functools.partial(
    pl.kernel,
    out_type=jax.ShapeDtypeStruct((BATCH,), jnp.float32),
    mesh=plsc.VectorSubcoreMesh(core_axis_name="c", subcore_axis_name="s"),
    compiler_params=pltpu.CompilerParams(needs_layout_passes=False),
    scratch_types=[
        pltpu.VMEM((NCHUNK, CHUNK), jnp.int32),
        pltpu.VMEM((NCHUNK, CHUNK), jnp.int32),
        pltpu.VMEM((CHUNK, K), jnp.float32),
        pltpu.VMEM((CHUNK, K), jnp.float32),
        pltpu.VMEM((CHUNK, K), jnp.float32),
        pltpu.VMEM((CHUNK, K), jnp.float32),
        pltpu.VMEM((BPW,), jnp.float32),
        pltpu.VMEM((BPW,), jnp.float32),
        pltpu.VMEM((16,), jnp.float32),
        pltpu.VMEM((BPW,), jnp.float32),
        pltpu.SemaphoreType.DMA,
        pltpu.SemaphoreType.DMA,
        pltpu.SemaphoreType.DMA,
    ],
)(_mf_body)


def kernel(user, item, Gu, Gi, Bu, Bi, Mu):
    mu16 = jnp.broadcast_to(jnp.reshape(Mu, (1,)), (16,))
    return _mf_sc(user.astype(jnp.int32), item.astype(jnp.int32), Gu, Gi,
                  jnp.reshape(Bu, (-1,)), jnp.reshape(Bi, (-1,)), mu16)


# speedup vs baseline: 1.7574x; 1.2644x over previous
import functools

import jax
import jax.numpy as jnp
from jax import lax
from jax.experimental import pallas as pl
from jax.experimental.pallas import tpu as pltpu
from jax.experimental.pallas import tpu_sc as plsc

BATCH = 16384
K = 128
NW = 32
BPW = BATCH // NW
NCHUNK = 4
CHUNK = BPW // NCHUNK
GROUPS = CHUNK // 16
UNR = 16


def _mf_body(user_hbm, item_hbm, gu_hbm, gi_hbm, bu_hbm, bi_hbm, mu_hbm,
             out_hbm, idx_u, idx_i, ru0, ru1, ri0, ri1, bu_v, bi_v, mu_v,
             out_v, sem0, sem1, semb):
    c = lax.axis_index("c")
    s = lax.axis_index("s")
    wid = s * 2 + c
    base = wid * BPW

    pltpu.sync_copy(mu_hbm, mu_v)
    for ch in range(NCHUNK):
        pltpu.sync_copy(user_hbm.at[pl.ds(base + ch * CHUNK, CHUNK)],
                        idx_u.at[ch])
        pltpu.sync_copy(item_hbm.at[pl.ds(base + ch * CHUNK, CHUNK)],
                        idx_i.at[ch])
    mu = mu_v[...]

    bias_copies = []
    for ch in range(NCHUNK):
        bias_copies.append(pltpu.async_copy(
            bu_hbm.at[idx_u.at[ch]], bu_v.at[pl.ds(ch * CHUNK, CHUNK)], semb))
        bias_copies.append(pltpu.async_copy(
            bi_hbm.at[idx_i.at[ch]], bi_v.at[pl.ds(ch * CHUNK, CHUNK)], semb))

    rbufs = [(ru0, ri0, sem0), (ru1, ri1, sem1)]

    def fire(ch):
        ru, ri, sem = rbufs[ch % 2]
        return (pltpu.async_copy(gu_hbm.at[idx_u.at[ch]], ru, sem),
                pltpu.async_copy(gi_hbm.at[idx_i.at[ch]], ri, sem))

    pending = fire(0)
    for ch in range(NCHUNK):
        nxt = fire(ch + 1) if ch + 1 < NCHUNK else None
        pending[0].wait()
        pending[1].wait()
        ru, ri, _ = rbufs[ch % 2]

        def group_body(g, carry, ru=ru, ri=ri, ch=ch):
            lane = lax.iota(jnp.int32, 16)
            rid = g * 16 + lane

            def k_body(j, accs, rid=rid, ru=ru, ri=ri):
                accs = list(accs)
                cb = rid + j * UNR + rid * 0
                for t in range(UNR):
                    col = (cb + t) & (K - 1)
                    uu = plsc.load_gather(ru, [rid, col])
                    vv = plsc.load_gather(ri, [rid, col])
                    accs[t % 4] = accs[t % 4] + uu * vv
                return tuple(accs)

            z = jnp.zeros((16,), jnp.float32)
            accs = lax.fori_loop(0, K // UNR, k_body, (z, z, z, z))
            acc = (accs[0] + accs[1]) + (accs[2] + accs[3])
            out_v[pl.ds(ch * CHUNK + g * 16, 16)] = acc
            return carry

        lax.fori_loop(0, GROUPS, group_body, 0)
        pending = nxt

    for cp in bias_copies:
        cp.wait()

    def epi_body(g, carry):
        sl = pl.ds(g * 16, 16)
        out_v[sl] = out_v[sl] + bu_v[sl] + bi_v[sl] + mu
        return carry

    lax.fori_loop(0, BPW // 16, epi_body, 0)
    pltpu.sync_copy(out_v, out_hbm.at[pl.ds(base, BPW)])


_mf_sc = functools.partial(
    pl.kernel,
    out_type=jax.ShapeDtypeStruct((BATCH,), jnp.float32),
    mesh=plsc.VectorSubcoreMesh(core_axis_name="c", subcore_axis_name="s"),
    compiler_params=pltpu.CompilerParams(needs_layout_passes=False),
    scratch_types=[
        pltpu.VMEM((NCHUNK, CHUNK), jnp.int32),
        pltpu.VMEM((NCHUNK, CHUNK), jnp.int32),
        pltpu.VMEM((CHUNK, K), jnp.float32),
        pltpu.VMEM((CHUNK, K), jnp.float32),
        pltpu.VMEM((CHUNK, K), jnp.float32),
        pltpu.VMEM((CHUNK, K), jnp.float32),
        pltpu.VMEM((BPW,), jnp.float32),
        pltpu.VMEM((BPW,), jnp.float32),
        pltpu.VMEM((16,), jnp.float32),
        pltpu.VMEM((BPW,), jnp.float32),
        pltpu.SemaphoreType.DMA,
        pltpu.SemaphoreType.DMA,
        pltpu.SemaphoreType.DMA,
    ],
)(_mf_body)


def kernel(user, item, Gu, Gi, Bu, Bi, Mu):
    mu16 = jnp.broadcast_to(jnp.reshape(Mu, (1,)), (16,))
    return _mf_sc(user.astype(jnp.int32), item.astype(jnp.int32), Gu, Gi,
                  jnp.reshape(Bu, (-1,)), jnp.reshape(Bi, (-1,)), mu16)
